# baseline (speedup 1.0000x reference)
import jax
import jax.numpy as jnp
from jax import lax
from jax.experimental import pallas as pl
from jax.experimental.pallas import tpu as pltpu
from jax.experimental.pallas import tpu_sc as plsc

B = 16384
F = 26
FIELD = 38461
NC, NS, L = 2, 16, 16
NW = NC * NS
BPW = B // NW
E = BPW * F
VPF = BPW // L
CHUNK = 512
UNROLL = 4
CPF = BPW // CHUNK


def _idx_body(xt_hbm, idx_hbm, xv, idxv, sem):
    wid = lax.axis_index("s") * NC + lax.axis_index("c")
    bbase = wid * BPW

    for f in range(F):
        pltpu.async_copy(
            xt_hbm.at[f, pl.ds(bbase, BPW)],
            xv.at[pl.ds(f * BPW, BPW)],
            sem,
        )
    pltpu.make_async_copy(xt_hbm.at[0, pl.ds(0, E)], xv, sem).wait()

    def build_field(f, carry):
        fbase = pl.multiple_of(f * BPW, BPW)
        off_vec = jnp.full((L,), f * FIELD, dtype=jnp.int32)

        def build_vec(v, carry2):
            for u in range(UNROLL):
                off = pl.multiple_of(fbase + (v * UNROLL + u) * L, L)
                idxv[pl.ds(off, L)] = xv[pl.ds(off, L)] + off_vec
            return carry2

        lax.fori_loop(0, VPF // UNROLL, build_vec, 0)
        return carry

    lax.fori_loop(0, F, build_field, 0)
    pltpu.sync_copy(idxv, idx_hbm.at[pl.ds(wid * E, E)])


def _gather_body(idx_hbm, tbl_hbm, bias_hbm, out_hbm,
                 idxv, rows, outv, biasv, sem):
    wid = lax.axis_index("s") * NC + lax.axis_index("c")

    pltpu.async_copy(idx_hbm.at[pl.ds(wid * E, E)], idxv, sem)
    pltpu.sync_copy(bias_hbm, biasv)
    pltpu.make_async_copy(idx_hbm.at[pl.ds(0, E)], idxv, sem).wait()

    def fire(j, carry):
        off = pl.multiple_of(j * CHUNK, CHUNK)
        pltpu.async_copy(
            tbl_hbm.at[idxv.at[pl.ds(off, CHUNK)]],
            rows.at[pl.ds(off, CHUNK)],
            sem,
        )
        return carry

    lax.fori_loop(0, F * CPF, fire, 0)
    pltpu.make_async_copy(tbl_hbm.at[pl.ds(0, E)], rows, sem).wait()

    bias16 = biasv[...]

    def reduce(c, carry):
        cbase = pl.multiple_of(c * L, L)
        acc = bias16
        for f in range(F):
            acc = acc + rows[pl.ds(f * BPW + cbase, L)]
        outv[pl.ds(cbase, L)] = acc
        return carry

    lax.fori_loop(0, VPF, reduce, 0)

    pltpu.sync_copy(outv, out_hbm.at[pl.ds(wid * BPW, BPW)])


def kernel(x, table, bias):
    xt = x.astype(jnp.int32).T
    tbl = jnp.pad(table, ((0, 462), (0, 0))).reshape(-1)
    bias16 = jnp.broadcast_to(bias.astype(jnp.float32), (L,))
    mesh = plsc.VectorSubcoreMesh(
        core_axis_name="c", subcore_axis_name="s",
        num_cores=NC, num_subcores=NS,
    )
    params = pltpu.CompilerParams(needs_layout_passes=False)
    idx_all = pl.kernel(
        _idx_body,
        out_type=jax.ShapeDtypeStruct((B * F,), jnp.int32),
        mesh=mesh,
        compiler_params=params,
        scratch_types=[
            pltpu.VMEM((E,), jnp.int32),
            pltpu.VMEM((E,), jnp.int32),
            pltpu.SemaphoreType.DMA,
        ],
    )(xt)
    out = pl.kernel(
        _gather_body,
        out_type=jax.ShapeDtypeStruct((B,), jnp.float32),
        mesh=mesh,
        compiler_params=params,
        scratch_types=[
            pltpu.VMEM((E,), jnp.int32),
            pltpu.VMEM((E,), jnp.float32),
            pltpu.VMEM((BPW,), jnp.float32),
            pltpu.VMEM((L,), jnp.float32),
            pltpu.SemaphoreType.DMA,
        ],
    )(idx_all, tbl, bias16)
    return out.reshape(B, 1)


# speedup vs baseline: 1.0393x; 1.0393x over previous
import jax
import jax.numpy as jnp
from jax import lax
from jax.experimental import pallas as pl
from jax.experimental.pallas import tpu as pltpu
from jax.experimental.pallas import tpu_sc as plsc

B = 16384
F = 26
FIELD = 38461
NC, NS, L = 2, 16, 16
NW = NC * NS
BPW = B // NW
E = BPW * F
VPF = BPW // L
CHUNK = 512
NCH = E // CHUNK


def _sc_body(xt_hbm, tbl_hbm, bias_hbm, out_hbm, xv, idxv, rows, outv, biasv, sem):
    wid = lax.axis_index("s") * NC + lax.axis_index("c")
    bbase = wid * BPW

    for f in range(F):
        pltpu.async_copy(
            xt_hbm.at[f, pl.ds(bbase, BPW)],
            xv.at[pl.ds(f * BPW, BPW)],
            sem,
        )
    pltpu.sync_copy(bias_hbm, biasv)
    pltpu.make_async_copy(xt_hbm.at[0, pl.ds(0, E)], xv, sem).wait()

    UNROLL = 4
    CPF = BPW // CHUNK

    def build_field(f, carry):
        fbase = pl.multiple_of(f * BPW, BPW)
        off_vec = jnp.full((L,), f * FIELD, dtype=jnp.int32)

        def build_vec(v, carry2):
            for u in range(UNROLL):
                off = pl.multiple_of(fbase + (v * UNROLL + u) * L, L)
                idxv[pl.ds(off, L)] = xv[pl.ds(off, L)] + off_vec
            return carry2

        lax.fori_loop(0, VPF // UNROLL, build_vec, 0)
        for j in range(CPF):
            off = pl.multiple_of(fbase + j * CHUNK, CHUNK)
            pltpu.async_copy(
                tbl_hbm.at[idxv.at[pl.ds(off, CHUNK)]],
                rows.at[pl.ds(off, CHUNK)],
                sem,
            )
        return carry

    lax.fori_loop(0, F, build_field, 0)
    pltpu.make_async_copy(tbl_hbm.at[pl.ds(0, E)], rows, sem).wait()

    bias16 = biasv[...]

    def reduce(c, carry):
        cbase = pl.multiple_of(c * L, L)
        acc = bias16
        for f in range(F):
            acc = acc + rows[pl.ds(f * BPW + cbase, L)]
        outv[pl.ds(cbase, L)] = acc
        return carry

    lax.fori_loop(0, VPF, reduce, 0)

    pltpu.sync_copy(outv, out_hbm.at[pl.ds(wid * BPW, BPW)])


def kernel(x, table, bias):
    xt = x.astype(jnp.int32).T
    tbl = jnp.concatenate(
        [table, jnp.zeros((462, 1), jnp.float32)], axis=0).reshape(-1)
    bias16 = jnp.broadcast_to(bias.astype(jnp.float32), (L,))
    mesh = plsc.VectorSubcoreMesh(
        core_axis_name="c", subcore_axis_name="s",
        num_cores=NC, num_subcores=NS,
    )
    out = pl.kernel(
        _sc_body,
        out_type=jax.ShapeDtypeStruct((B,), jnp.float32),
        mesh=mesh,
        compiler_params=pltpu.CompilerParams(needs_layout_passes=False),
        scratch_types=[
            pltpu.VMEM((E,), jnp.int32),
            pltpu.VMEM((E,), jnp.int32),
            pltpu.VMEM((E,), jnp.float32),
            pltpu.VMEM((BPW,), jnp.float32),
            pltpu.VMEM((L,), jnp.float32),
            pltpu.SemaphoreType.DMA,
        ],
    )(xt, tbl, bias16)
    return out.reshape(B, 1)


# speedup vs baseline: 1.0452x; 1.0056x over previous
import jax
import jax.numpy as jnp
from jax import lax
from jax.experimental import pallas as pl
from jax.experimental.pallas import tpu as pltpu
from jax.experimental.pallas import tpu_sc as plsc

B = 16384
F = 26
FIELD = 38461
NC, NS, L = 2, 16, 16
NW = NC * NS
BPW = B // NW
E = BPW * F
VPF = BPW // L
CHUNK = 512
NCH = E // CHUNK


def _sc_body(xt_hbm, tbl_hbm, bias_hbm, out_hbm, xv, idxv, rows, outv, biasv,
             sem, sem0, semx):
    wid = lax.axis_index("s") * NC + lax.axis_index("c")
    bbase = wid * BPW

    HEAD = 4
    for f in range(F):
        pltpu.async_copy(
            xt_hbm.at[f, pl.ds(bbase, BPW)],
            xv.at[pl.ds(f * BPW, BPW)],
            sem0 if f < HEAD else semx,
        )
    pltpu.sync_copy(bias_hbm, biasv)

    UNROLL = 4
    CPF = BPW // CHUNK

    def build_field(f, carry):
        fbase = pl.multiple_of(f * BPW, BPW)
        off_vec = jnp.full((L,), f * FIELD, dtype=jnp.int32)

        def build_vec(v, carry2):
            for u in range(UNROLL):
                off = pl.multiple_of(fbase + (v * UNROLL + u) * L, L)
                idxv[pl.ds(off, L)] = xv[pl.ds(off, L)] + off_vec
            return carry2

        lax.fori_loop(0, VPF // UNROLL, build_vec, 0)
        for j in range(CPF):
            off = pl.multiple_of(fbase + j * CHUNK, CHUNK)
            pltpu.async_copy(
                tbl_hbm.at[idxv.at[pl.ds(off, CHUNK)]],
                rows.at[pl.ds(off, CHUNK)],
                sem,
            )
        return carry

    pltpu.make_async_copy(xt_hbm.at[0, pl.ds(0, HEAD * BPW)],
                          xv.at[pl.ds(0, HEAD * BPW)], sem0).wait()
    lax.fori_loop(0, HEAD, build_field, 0)
    pltpu.make_async_copy(xt_hbm.at[0, pl.ds(0, (F - HEAD) * BPW)],
                          xv.at[pl.ds(0, (F - HEAD) * BPW)], semx).wait()
    lax.fori_loop(HEAD, F, build_field, 0)
    pltpu.make_async_copy(tbl_hbm.at[pl.ds(0, E)], rows, sem).wait()

    bias16 = biasv[...]

    def reduce(c, carry):
        cbase = pl.multiple_of(c * L, L)
        acc = bias16
        for f in range(F):
            acc = acc + rows[pl.ds(f * BPW + cbase, L)]
        outv[pl.ds(cbase, L)] = acc
        return carry

    lax.fori_loop(0, VPF, reduce, 0)

    pltpu.sync_copy(outv, out_hbm.at[pl.ds(wid * BPW, BPW)])


def kernel(x, table, bias):
    xt = x.astype(jnp.int32).T
    tbl = jnp.pad(table, ((0, 462), (0, 0))).reshape(-1)
    bias16 = jnp.broadcast_to(bias.astype(jnp.float32), (L,))
    mesh = plsc.VectorSubcoreMesh(
        core_axis_name="c", subcore_axis_name="s",
        num_cores=NC, num_subcores=NS,
    )
    out = pl.kernel(
        _sc_body,
        out_type=jax.ShapeDtypeStruct((B,), jnp.float32),
        mesh=mesh,
        compiler_params=pltpu.CompilerParams(needs_layout_passes=False),
        scratch_types=[
            pltpu.VMEM((E,), jnp.int32),
            pltpu.VMEM((E,), jnp.int32),
            pltpu.VMEM((E,), jnp.float32),
            pltpu.VMEM((BPW,), jnp.float32),
            pltpu.VMEM((L,), jnp.float32),
            pltpu.SemaphoreType.DMA,
            pltpu.SemaphoreType.DMA,
            pltpu.SemaphoreType.DMA,
        ],
    )(xt, tbl, bias16)
    return out.reshape(B, 1)


# speedup vs baseline: 1.1842x; 1.1330x over previous
import jax
import jax.numpy as jnp
from jax import lax
from jax.experimental import pallas as pl
from jax.experimental.pallas import tpu as pltpu
from jax.experimental.pallas import tpu_sc as plsc

B = 16384
F = 26
FIELD = 38461
NC, NS, L = 2, 16, 16
NW = NC * NS
BPW = B // NW
WIN = 38472
VECS = B // L
UNROLL = 4


def _lookup_body(xt_hbm, tbl_hbm, part_hbm, xcol, win, partial, sem):
    wid = lax.axis_index("s") * NC + lax.axis_index("c")

    @pl.when(wid < F)
    def _():
        start = pl.multiple_of((wid * FIELD) & ~7, 8)
        corr = wid * FIELD - start
        pltpu.async_copy(tbl_hbm.at[pl.ds(start, WIN)], win, sem)
        pltpu.async_copy(xt_hbm.at[wid, pl.ds(0, B)], xcol, sem)
        pltpu.make_async_copy(tbl_hbm.at[pl.ds(0, WIN)], win, sem).wait()
        pltpu.make_async_copy(xt_hbm.at[0, pl.ds(0, B)], xcol, sem).wait()

        corr16 = jnp.full((L,), corr, dtype=jnp.int32)

        def lookup(i, carry):
            for u in range(UNROLL):
                off = pl.multiple_of((i * UNROLL + u) * L, L)
                idx16 = xcol[pl.ds(off, L)] + corr16
                partial[pl.ds(off, L)] = plsc.load_gather(win, [idx16])
            return carry

        lax.fori_loop(0, VECS // UNROLL, lookup, 0)
        pltpu.sync_copy(partial, part_hbm.at[wid, pl.ds(0, B)])


def _reduce_body(part_hbm, bias_hbm, out_hbm, pv, outv, biasv, sem):
    wid = lax.axis_index("s") * NC + lax.axis_index("c")
    bbase = wid * BPW

    for f in range(F):
        pltpu.async_copy(
            part_hbm.at[f, pl.ds(bbase, BPW)],
            pv.at[pl.ds(f * BPW, BPW)],
            sem,
        )
    pltpu.sync_copy(bias_hbm, biasv)
    pltpu.make_async_copy(part_hbm.at[0, pl.ds(0, F * BPW)], pv, sem).wait()

    bias16 = biasv[...]

    def reduce(c, carry):
        cbase = pl.multiple_of(c * L, L)
        acc = bias16
        for f in range(F):
            acc = acc + pv[pl.ds(f * BPW + cbase, L)]
        outv[pl.ds(cbase, L)] = acc
        return carry

    lax.fori_loop(0, BPW // L, reduce, 0)

    pltpu.sync_copy(outv, out_hbm.at[pl.ds(wid * BPW, BPW)])


def kernel(x, table, bias):
    xt = x.astype(jnp.int32).T
    tbl = jnp.pad(table, ((0, 462), (0, 0))).reshape(-1)
    bias16 = jnp.broadcast_to(bias.astype(jnp.float32), (L,))
    mesh = plsc.VectorSubcoreMesh(
        core_axis_name="c", subcore_axis_name="s",
        num_cores=NC, num_subcores=NS,
    )
    params = pltpu.CompilerParams(needs_layout_passes=False)
    partials = pl.kernel(
        _lookup_body,
        out_type=jax.ShapeDtypeStruct((F, B), jnp.float32),
        mesh=mesh,
        compiler_params=params,
        scratch_types=[
            pltpu.VMEM((B,), jnp.int32),
            pltpu.VMEM((WIN,), jnp.float32),
            pltpu.VMEM((B,), jnp.float32),
            pltpu.SemaphoreType.DMA,
        ],
    )(xt, tbl)
    out = pl.kernel(
        _reduce_body,
        out_type=jax.ShapeDtypeStruct((B,), jnp.float32),
        mesh=mesh,
        compiler_params=params,
        scratch_types=[
            pltpu.VMEM((F * BPW,), jnp.float32),
            pltpu.VMEM((BPW,), jnp.float32),
            pltpu.VMEM((L,), jnp.float32),
            pltpu.SemaphoreType.DMA,
        ],
    )(partials, bias16)
    return out.reshape(B, 1)
